# baseline (speedup 1.0000x reference)
import functools

import jax
import jax.numpy as jnp
from jax import lax
from jax.experimental import pallas as pl
from jax.experimental.pallas import tpu as pltpu
from jax.experimental.pallas import tpu_sc as plsc

N = 10000
E = 320000
D = 128
R = 4
B = 4096
PH = 128

R1 = R + 1
NC = 2
NS = 16
NW = NC * NS
EPW = E // NW
EPT = E // NS
CE = 80
NCH = EPW // CE
DCH = EPT // CE
SCK = 2000
SCP = SCK // CE
NSC = EPW // SCK
DEGP = 40960
DSL = DEGP // NS
NP = 10240
NPT = NP // NS
BN = 1000
NB = N // BN
BPW = B // NW

_f32 = jnp.float32
_i32 = jnp.int32


def _mesh():
    return plsc.VectorSubcoreMesh(
        core_axis_name="c", subcore_axis_name="s", num_cores=NC, num_subcores=NS
    )


def _perm(acc):
    u = jax.lax.bitcast_convert_type(acc, jnp.uint32)
    bb = (u + jnp.uint32(0x7FFF) + ((u >> 16) & jnp.uint32(1))) >> 16
    v4 = bb.reshape(BN, 4, 2, 16)
    w = v4[:, :, 0, :] | (v4[:, :, 1, :] << 16)
    return jax.lax.bitcast_convert_type(w.reshape(BN, D // 2), jnp.int32)


def _mm_body(x_ref, w_ref, o_tab, o_self):
    r = pl.program_id(1)
    acc = jnp.dot(x_ref[...], w_ref[0], preferred_element_type=_f32)
    o_tab[0] = _perm(acc)

    @pl.when(r == 0)
    def _():
        o_self[...] = acc


def _tc_tables(x, w):
    return pl.pallas_call(
        _mm_body,
        grid=(NB, R1),
        in_specs=[
            pl.BlockSpec((BN, D), lambda i, r: (i, 0)),
            pl.BlockSpec((1, D, D), lambda i, r: (r, 0, 0)),
        ],
        out_specs=[
            pl.BlockSpec((1, BN, D // 2), lambda i, r: (r, i, 0)),
            pl.BlockSpec((BN, D), lambda i, r: (i, 0)),
        ],
        out_shape=[
            jax.ShapeDtypeStruct((R1, N, D // 2), jnp.int32),
            jax.ShapeDtypeStruct((N, D), _f32),
        ],
    )(x, w)


def _mm_fused_body(p0_ref, p1_ref, s0_ref, b_ref, w_ref, o_tab, o_self):
    r = pl.program_id(1)
    x = jnp.maximum(p0_ref[...] + p1_ref[...] + s0_ref[...] + b_ref[...], 0.0)
    acc = jnp.dot(x, w_ref[0], preferred_element_type=_f32)
    o_tab[0] = _perm(acc)

    @pl.when(r == 0)
    def _():
        o_self[...] = acc


def _tc_tables_fused(p0, p1, s0, b, w):
    return pl.pallas_call(
        _mm_fused_body,
        grid=(NB, R1),
        in_specs=[
            pl.BlockSpec((BN, D), lambda i, r: (i, 0)),
            pl.BlockSpec((BN, D), lambda i, r: (i, 0)),
            pl.BlockSpec((BN, D), lambda i, r: (i, 0)),
            pl.BlockSpec((1, D), lambda i, r: (0, 0)),
            pl.BlockSpec((1, D, D), lambda i, r: (r, 0, 0)),
        ],
        out_specs=[
            pl.BlockSpec((1, BN, D // 2), lambda i, r: (r, i, 0)),
            pl.BlockSpec((BN, D), lambda i, r: (i, 0)),
        ],
        out_shape=[
            jax.ShapeDtypeStruct((R1, N, D // 2), jnp.int32),
            jax.ShapeDtypeStruct((N, D), _f32),
        ],
    )(p0, p1, s0, b, w)


def _sum_body(p0_ref, p1_ref, s0_ref, b_ref, o_ref):
    o_ref[...] = p0_ref[...] + p1_ref[...] + s0_ref[...] + b_ref[...]


def _tc_sum(p0, p1, s0, b):
    return pl.pallas_call(
        _sum_body,
        grid=(NB,),
        in_specs=[
            pl.BlockSpec((BN, D), lambda i: (i, 0)),
            pl.BlockSpec((BN, D), lambda i: (i, 0)),
            pl.BlockSpec((BN, D), lambda i: (i, 0)),
            pl.BlockSpec((1, D), lambda i: (0, 0)),
        ],
        out_specs=pl.BlockSpec((BN, D), lambda i: (i, 0)),
        out_shape=jax.ShapeDtypeStruct((N, D), _f32),
    )(p0, p1, s0, b)


def _pair_body(dr_ref, di_ref, w1_ref, b1_ref, w2_ref, b2_ref, o_ref):
    dr = dr_ref[...]
    di = di_ref[...]
    h = (
        jnp.dot(dr, w1_ref[0:D, :], preferred_element_type=_f32)
        + jnp.dot(di, w1_ref[D : 2 * D, :], preferred_element_type=_f32)
        + jnp.dot(dr * di, w1_ref[2 * D : 3 * D, :], preferred_element_type=_f32)
        + b1_ref[...]
    )
    h = jnp.maximum(h, 0.0)
    o_ref[...] = jnp.dot(h, w2_ref[...], preferred_element_type=_f32) + b2_ref[...]


def _tc_pair(dr, di, w1, b1, w2, b2):
    BB = 512
    return pl.pallas_call(
        _pair_body,
        grid=(B // BB,),
        in_specs=[
            pl.BlockSpec((BB, D), lambda i: (i, 0)),
            pl.BlockSpec((BB, D), lambda i: (i, 0)),
            pl.BlockSpec((3 * D, PH), lambda i: (0, 0)),
            pl.BlockSpec((1, PH), lambda i: (0, 0)),
            pl.BlockSpec((PH, 1), lambda i: (0, 0)),
            pl.BlockSpec((1, 1), lambda i: (0, 0)),
        ],
        out_specs=pl.BlockSpec((BB, 1), lambda i: (i, 0)),
        out_shape=jax.ShapeDtypeStruct((B, 1), _f32),
    )(dr, di, w1, b1, w2, b2)


_Z16 = lambda: jnp.zeros((16,), _f32)


def _zero_rows(zrow_v):
    z = _Z16()

    def row(i, _):
        for jj in range(8):
            zrow_v[i, pl.ds(jj * 16, 16)] = z
        return 0

    lax.fori_loop(0, 16, row, 0)


def _scale_rows(msg_v, scl_v, count):
    def srow(e, _):
        bc = plsc.load_gather(scl_v, [jnp.broadcast_to(e, (16,)).astype(_i32)])
        for jj in range(8):
            msg_v[e, pl.ds(jj * 16, 16)] = msg_v[e, pl.ds(jj * 16, 16)] * bc
        return 0

    lax.fori_loop(0, count, srow, 0)


NSC_D = EPT // SCK


def _sc_degree_body(edst, et, stab, deg_sh, dbuf_v, edstS, etS, sixS2,
                    ones_v, semD):
    c = lax.axis_index("c")
    s = lax.axis_index("s")
    z = _Z16()
    one = jnp.ones((16,), _f32)

    def zdeg(i, _):
        dbuf_v[pl.ds(i * 16, 16)] = z
        return 0

    lax.fori_loop(0, DSL // 16, zdeg, 0)
    for g in range(CE // 16):
        ones_v[pl.ds(g * 16, 16)] = one
    pltpu.sync_copy(dbuf_v, deg_sh.at[pl.ds(s * DSL, DSL)])
    plsc.subcore_barrier()

    def sblk(S, _):
        sbase = s * EPT + S * SCK
        pltpu.sync_copy(edst.at[pl.ds(sbase, SCK)], edstS)
        pltpu.sync_copy(et.at[pl.ds(sbase, SCK)], etS)
        for p in range(SCP):
            for g in range(CE // 16):
                sl = pl.ds(p * CE + g * 16, 16)
                sixS2[p, pl.ds(g * 16, 16)] = etS[sl] * N + edstS[sl]
        descs = [
            pltpu.async_copy(ones_v, deg_sh.at[sixS2.at[p]], semD, add=True)
            for p in range(SCP)
        ]
        for d in descs:
            d.wait()
        return 0

    lax.fori_loop(0, NSC_D, sblk, 0)
    plsc.subcore_barrier()

    pltpu.sync_copy(deg_sh.at[pl.ds(s * DSL, DSL)], dbuf_v)

    def recip(i, _):
        sl = pl.ds(i * 16, 16)
        dbuf_v[sl] = 1.0 / jnp.maximum(dbuf_v[sl], 1.0)
        return 0

    lax.fori_loop(0, DSL // 16, recip, 0)

    @pl.when(c == 0)
    def _():
        pltpu.sync_copy(dbuf_v, stab.at[pl.ds(s * DSL, DSL)])


def _sc_degree(edst, et):
    f = pl.kernel(
        _sc_degree_body,
        out_type=jax.ShapeDtypeStruct((DEGP,), _f32),
        mesh=_mesh(),
        compiler_params=pltpu.CompilerParams(needs_layout_passes=False),
        scratch_types=[
            pltpu.VMEM_SHARED((DEGP,), _f32),
            pltpu.VMEM((DSL,), _f32),
            pltpu.VMEM((SCK,), _i32),
            pltpu.VMEM((SCK,), _i32),
            pltpu.VMEM((SCP, CE), _i32),
            pltpu.VMEM((CE,), _f32),
            pltpu.SemaphoreType.DMA,
        ],
    )
    return f(edst, et)


def _scale_rows_bf16(msg_bf, msgF, sclS, off):
    @plsc.parallel_loop(0, CE, unroll=2)
    def srow(e):
        bc = plsc.load_gather(sclS, [jnp.broadcast_to(off + e, (16,)).astype(_i32)])
        for j in range(4):
            v = plsc.bitcast(msg_bf[e, pl.ds(j * 16, 16)], jnp.bfloat16)
            a, b = plsc.unpack(v, format=plsc.PackFormat.INTERLEAVED)
            msgF[e, pl.ds(j * 32, 16)] = a * bc
            msgF[e, pl.ds(j * 32 + 16, 16)] = b * bc


def _msg_superchunk(xwtab, edstS, gixS, sclS, acc_sh, dsts, msgs, msgFs,
                    semG, semW):
    ns = len(msgs)
    nd = len(dsts)
    nf = len(msgFs)
    gds = {}
    scs = {}

    def fire(p):
        for g in range(CE // 16):
            dsts[p % nd][pl.ds(g * 16, 16)] = edstS[pl.ds(p * CE + g * 16, 16)]
        gds[p] = pltpu.async_copy(
            xwtab.at[gixS.at[pl.ds(p * CE, CE)]], msgs[p % ns], semG
        )

    for p in range(ns - 1):
        fire(p)
    for p in range(SCP):
        if p - nf >= 0:
            scs[p - nf].wait()
        gds[p].wait()
        _scale_rows_bf16(msgs[p % ns], msgFs[p % nf], sclS, p * CE)
        scs[p] = pltpu.async_copy(
            msgFs[p % nf], acc_sh.at[dsts[p % nd]], semW, add=True
        )
        q = p + ns - 1
        if q < SCP:
            fire(q)
    for p in range(max(0, SCP - nf), SCP):
        scs[p].wait()


def _zero_acc(zrow_v, acc_sh, s):
    _zero_rows(zrow_v)

    def zacc(k, _):
        pltpu.sync_copy(zrow_v, acc_sh.at[pl.ds(s * NPT + k * 16, 16), :])
        return 0

    lax.fori_loop(0, NPT // 16, zacc, 0)


def _writeback(acc_sh, partial, c, s):
    pltpu.sync_copy(
        acc_sh.at[pl.ds(s * NPT, NPT), :], partial.at[c, pl.ds(s * NPT, NPT), :]
    )


def _sc_layer1_body(
    xwtab, esrc, edst, et, stab,
    partial, gidx_out, scale_out,
    acc_sh,
    zrow_v,
    srcS, edstS, etS, gixS, sixS, sclS,
    dst0, dst1, dst2, dst3, msg0, msg1, msgF0, msgF1, semG, semW, semS,
):
    c = lax.axis_index("c")
    s = lax.axis_index("s")
    w = c * NS + s

    _zero_acc(zrow_v, acc_sh, s)
    plsc.subcore_barrier()

    def sblk(S, _):
        sbase = w * EPW + S * SCK
        pltpu.sync_copy(esrc.at[pl.ds(sbase, SCK)], srcS)
        pltpu.sync_copy(et.at[pl.ds(sbase, SCK)], etS)
        pltpu.sync_copy(edst.at[pl.ds(sbase, SCK)], edstS)
        for i in range(SCK // 16):
            sl = pl.ds(i * 16, 16)
            t16 = etS[sl]
            gixS[sl] = (t16 + 1) * N + srcS[sl]
            sixS[sl] = t16 * N + edstS[sl]
        descs = [
            pltpu.async_copy(
                stab.at[sixS.at[pl.ds(p * CE, CE)]],
                sclS.at[pl.ds(p * CE, CE)], semS,
            )
            for p in range(SCP)
        ]
        for d in descs:
            d.wait()
        pltpu.sync_copy(gixS, gidx_out.at[pl.ds(sbase, SCK)])
        pltpu.sync_copy(sclS, scale_out.at[pl.ds(sbase, SCK)])
        _msg_superchunk(xwtab, edstS, gixS, sclS, acc_sh,
                        (dst0, dst1, dst2, dst3), (msg0, msg1),
                        (msgF0, msgF1), semG, semW)
        return 0

    lax.fori_loop(0, NSC, sblk, 0)
    plsc.subcore_barrier()
    _writeback(acc_sh, partial, c, s)


def _sc_layer1(xwtab_flat, esrc, edst, et, stab):
    f = pl.kernel(
        _sc_layer1_body,
        out_type=(
            jax.ShapeDtypeStruct((NC, NP, D), _f32),
            jax.ShapeDtypeStruct((E,), _i32),
            jax.ShapeDtypeStruct((E,), _f32),
        ),
        mesh=_mesh(),
        compiler_params=pltpu.CompilerParams(
            needs_layout_passes=False, use_tc_tiling_on_sc=False),
        scratch_types=[
            pltpu.VMEM_SHARED((NP, D), _f32),
            pltpu.VMEM((16, D), _f32),
            pltpu.VMEM((SCK,), _i32),
            pltpu.VMEM((SCK,), _i32),
            pltpu.VMEM((SCK,), _i32),
            pltpu.VMEM((SCK,), _i32),
            pltpu.VMEM((SCK,), _i32),
            pltpu.VMEM((SCK,), _f32),
            pltpu.VMEM((CE,), _i32),
            pltpu.VMEM((CE,), _i32),
            pltpu.VMEM((CE,), _i32),
            pltpu.VMEM((CE,), _i32),
            pltpu.VMEM((CE, D // 2), _i32),
            pltpu.VMEM((CE, D // 2), _i32),
            pltpu.VMEM((CE, D), _f32),
            pltpu.VMEM((CE, D), _f32),
            pltpu.SemaphoreType.DMA,
            pltpu.SemaphoreType.DMA,
            pltpu.SemaphoreType.DMA,
        ],
    )
    return f(xwtab_flat, esrc, edst, et, stab)


def _sc_layer2_body(
    xwtab, edst, gidx, scale,
    partial,
    acc_sh, zrow_v, edstS, gixS, sclS,
    dst0, dst1, dst2, dst3, msg0, msg1, msg2, msgF0, msgF1, semG, semW,
):
    c = lax.axis_index("c")
    s = lax.axis_index("s")
    w = c * NS + s

    _zero_acc(zrow_v, acc_sh, s)
    plsc.subcore_barrier()

    def sblk(S, _):
        sbase = w * EPW + S * SCK
        pltpu.sync_copy(edst.at[pl.ds(sbase, SCK)], edstS)
        pltpu.sync_copy(gidx.at[pl.ds(sbase, SCK)], gixS)
        pltpu.sync_copy(scale.at[pl.ds(sbase, SCK)], sclS)
        _msg_superchunk(xwtab, edstS, gixS, sclS, acc_sh,
                        (dst0, dst1, dst2, dst3), (msg0, msg1, msg2),
                        (msgF0, msgF1), semG, semW)
        return 0

    lax.fori_loop(0, NSC, sblk, 0)
    plsc.subcore_barrier()
    _writeback(acc_sh, partial, c, s)


def _sc_layer2(xwtab_flat, edst, gidx, scale):
    f = pl.kernel(
        _sc_layer2_body,
        out_type=jax.ShapeDtypeStruct((NC, NP, D), _f32),
        mesh=_mesh(),
        compiler_params=pltpu.CompilerParams(
            needs_layout_passes=False, use_tc_tiling_on_sc=False),
        scratch_types=[
            pltpu.VMEM_SHARED((NP, D), _f32),
            pltpu.VMEM((16, D), _f32),
            pltpu.VMEM((SCK,), _i32),
            pltpu.VMEM((SCK,), _i32),
            pltpu.VMEM((SCK,), _f32),
            pltpu.VMEM((CE,), _i32),
            pltpu.VMEM((CE,), _i32),
            pltpu.VMEM((CE,), _i32),
            pltpu.VMEM((CE,), _i32),
            pltpu.VMEM((CE, D // 2), _i32),
            pltpu.VMEM((CE, D // 2), _i32),
            pltpu.VMEM((CE, D // 2), _i32),
            pltpu.VMEM((CE, D), _f32),
            pltpu.VMEM((CE, D), _f32),
            pltpu.SemaphoreType.DMA,
            pltpu.SemaphoreType.DMA,
        ],
    )
    return f(xwtab_flat, edst, gidx, scale)


def _sc_pair_gather_body(p0, p1, s1, b1, dridx, disidx, dr_out, di_out,
                         idx_v, a_v, b_v, bbuf, sem):
    c = lax.axis_index("c")
    s = lax.axis_index("s")
    w = c * NS + s
    base = w * BPW
    pltpu.sync_copy(b1, bbuf)

    for idxsrc, out in ((dridx, dr_out), (disidx, di_out)):
        pltpu.sync_copy(idxsrc.at[pl.ds(base, BPW)], idx_v)
        da = pltpu.async_copy(p0.at[idx_v], a_v, sem)
        db = pltpu.async_copy(p1.at[idx_v], b_v, sem)
        da.wait()
        db.wait()

        @plsc.parallel_loop(0, BPW, unroll=4)
        def _(e):
            for j in range(8):
                sl = pl.ds(j * 16, 16)
                a_v[e, sl] = a_v[e, sl] + b_v[e, sl]

        pltpu.async_copy(s1.at[idx_v], b_v, sem).wait()

        @plsc.parallel_loop(0, BPW, unroll=4)
        def _(e):
            for j in range(8):
                sl = pl.ds(j * 16, 16)
                a_v[e, sl] = a_v[e, sl] + b_v[e, sl] + bbuf[sl]

        pltpu.sync_copy(a_v, out.at[pl.ds(base, BPW), :])


def _sc_pair_gather(p0, p1, s1, b1, dridx, disidx):
    f = pl.kernel(
        _sc_pair_gather_body,
        out_type=(
            jax.ShapeDtypeStruct((B, D), _f32),
            jax.ShapeDtypeStruct((B, D), _f32),
        ),
        mesh=_mesh(),
        compiler_params=pltpu.CompilerParams(needs_layout_passes=False),
        scratch_types=[
            pltpu.VMEM((BPW,), _i32),
            pltpu.VMEM((BPW, D), _f32),
            pltpu.VMEM((BPW, D), _f32),
            pltpu.VMEM((D,), _f32),
            pltpu.SemaphoreType.DMA,
        ],
    )
    return f(p0, p1, s1, b1, dridx, disidx)


def kernel(emb, rel_w0, self_w0, b0, rel_w1, self_w1, b1, pW1, pb1, pW2, pb2,
           edge_index, edge_type, drug_index, disease_index):
    w0 = jnp.concatenate([self_w0[None], rel_w0], axis=0)
    w1 = jnp.concatenate([self_w1[None], rel_w1], axis=0)
    esrc = edge_index[0].astype(_i32)
    edst = edge_index[1].astype(_i32)
    et = edge_type.astype(_i32)

    stab = _sc_degree(edst, et)
    xw0tab, self0 = _tc_tables(emb, w0)
    part0, gidx, scale = _sc_layer1(xw0tab.reshape(R1 * N, D // 2), esrc, edst, et,
                                    stab)
    xw1tab, self1 = _tc_tables_fused(part0[0], part0[1], self0,
                                     b0.reshape(1, D), w1)
    part1 = _sc_layer2(xw1tab.reshape(R1 * N, D // 2), edst, gidx, scale)
    dr, di = _sc_pair_gather(part1[0], part1[1], self1, b1,
                             drug_index.astype(_i32), disease_index.astype(_i32))
    logits = _tc_pair(dr, di, pW1, pb1.reshape(1, PH), pW2, pb2.reshape(1, 1))
    return logits.reshape(B)


# speedup vs baseline: 2.1176x; 2.1176x over previous
import functools

import jax
import jax.numpy as jnp
from jax import lax
from jax.experimental import pallas as pl
from jax.experimental.pallas import tpu as pltpu
from jax.experimental.pallas import tpu_sc as plsc

N = 10000
E = 320000
D = 128
R = 4
B = 4096
PH = 128

R1 = R + 1
NC = 2
NS = 16
NW = NC * NS
EPW = E // NW
EPT = E // NS
CE = 80
NCH = EPW // CE
DCH = EPT // CE
SCK = 2000
SCP = SCK // CE
NSC = EPW // SCK
DEGP = 40960
DSL = DEGP // NS
NP = 10240
NPT = NP // NS
BN = 1000
NB = N // BN
BPW = B // NW

_f32 = jnp.float32
_i32 = jnp.int32


def _mesh():
    return plsc.VectorSubcoreMesh(
        core_axis_name="c", subcore_axis_name="s", num_cores=NC, num_subcores=NS
    )


def _mm_body(x_ref, w_ref, o_ref):
    res = jnp.dot(x_ref[...], w_ref[...], preferred_element_type=_f32)
    for r in range(R1):
        o_ref[r] = res[:, r * D:(r + 1) * D]


def _tc_tables(x, wcat):
    return pl.pallas_call(
        _mm_body,
        grid=(NB,),
        in_specs=[
            pl.BlockSpec((BN, D), lambda i: (i, 0)),
            pl.BlockSpec((D, R1 * D), lambda i: (0, 0)),
        ],
        out_specs=pl.BlockSpec((R1, BN, D), lambda i: (0, i, 0)),
        out_shape=jax.ShapeDtypeStruct((R1, N, D), _f32),
    )(x, wcat)


def _mm_fused_body(p0_ref, p1_ref, s0_ref, b_ref, w_ref, o_ref):
    x = jnp.maximum(p0_ref[...] + p1_ref[...] + s0_ref[...] + b_ref[...], 0.0)
    res = jnp.dot(x, w_ref[...], preferred_element_type=_f32)
    for r in range(R1):
        o_ref[r] = res[:, r * D:(r + 1) * D]


def _tc_tables_fused(p0, p1, s0, b, wcat):
    return pl.pallas_call(
        _mm_fused_body,
        grid=(NB,),
        in_specs=[
            pl.BlockSpec((BN, D), lambda i: (i, 0)),
            pl.BlockSpec((BN, D), lambda i: (i, 0)),
            pl.BlockSpec((BN, D), lambda i: (i, 0)),
            pl.BlockSpec((1, D), lambda i: (0, 0)),
            pl.BlockSpec((D, R1 * D), lambda i: (0, 0)),
        ],
        out_specs=pl.BlockSpec((R1, BN, D), lambda i: (0, i, 0)),
        out_shape=jax.ShapeDtypeStruct((R1, N, D), _f32),
    )(p0, p1, s0, b, wcat)


def _sum_body(p0_ref, p1_ref, s0_ref, b_ref, o_ref):
    o_ref[...] = p0_ref[...] + p1_ref[...] + s0_ref[...] + b_ref[...]


def _tc_sum(p0, p1, s0, b):
    return pl.pallas_call(
        _sum_body,
        grid=(NB,),
        in_specs=[
            pl.BlockSpec((BN, D), lambda i: (i, 0)),
            pl.BlockSpec((BN, D), lambda i: (i, 0)),
            pl.BlockSpec((BN, D), lambda i: (i, 0)),
            pl.BlockSpec((1, D), lambda i: (0, 0)),
        ],
        out_specs=pl.BlockSpec((BN, D), lambda i: (i, 0)),
        out_shape=jax.ShapeDtypeStruct((N, D), _f32),
    )(p0, p1, s0, b)


def _pair_body(dr_ref, di_ref, w1_ref, b1_ref, w2_ref, b2_ref, o_ref):
    dr = dr_ref[...]
    di = di_ref[...]
    h = (
        jnp.dot(dr, w1_ref[0:D, :], preferred_element_type=_f32)
        + jnp.dot(di, w1_ref[D : 2 * D, :], preferred_element_type=_f32)
        + jnp.dot(dr * di, w1_ref[2 * D : 3 * D, :], preferred_element_type=_f32)
        + b1_ref[...]
    )
    h = jnp.maximum(h, 0.0)
    o_ref[...] = jnp.dot(h, w2_ref[...], preferred_element_type=_f32) + b2_ref[...]


def _tc_pair(dr, di, w1, b1, w2, b2):
    BB = 512
    return pl.pallas_call(
        _pair_body,
        grid=(B // BB,),
        in_specs=[
            pl.BlockSpec((BB, D), lambda i: (i, 0)),
            pl.BlockSpec((BB, D), lambda i: (i, 0)),
            pl.BlockSpec((3 * D, PH), lambda i: (0, 0)),
            pl.BlockSpec((1, PH), lambda i: (0, 0)),
            pl.BlockSpec((PH, 1), lambda i: (0, 0)),
            pl.BlockSpec((1, 1), lambda i: (0, 0)),
        ],
        out_specs=pl.BlockSpec((BB, 1), lambda i: (i, 0)),
        out_shape=jax.ShapeDtypeStruct((B, 1), _f32),
    )(dr, di, w1, b1, w2, b2)


_Z16 = lambda: jnp.zeros((16,), _f32)


def _zero_rows(zrow_v):
    z = _Z16()

    def row(i, _):
        for jj in range(8):
            zrow_v[i, pl.ds(jj * 16, 16)] = z
        return 0

    lax.fori_loop(0, 16, row, 0)


def _scale_rows(msg_v, scl_v, count):
    def srow(e, _):
        bc = plsc.load_gather(scl_v, [jnp.broadcast_to(e, (16,)).astype(_i32)])
        for jj in range(8):
            msg_v[e, pl.ds(jj * 16, 16)] = msg_v[e, pl.ds(jj * 16, 16)] * bc
        return 0

    lax.fori_loop(0, count, srow, 0)


NSC_D = EPT // SCK


def _sc_degree_body(edst, et, stab, deg_sh, dbuf_v, edstS, etS, sixS2,
                    ones_v, semD):
    c = lax.axis_index("c")
    s = lax.axis_index("s")
    z = _Z16()
    one = jnp.ones((16,), _f32)

    def zdeg(i, _):
        dbuf_v[pl.ds(i * 16, 16)] = z
        return 0

    lax.fori_loop(0, DSL // 16, zdeg, 0)
    for g in range(CE // 16):
        ones_v[pl.ds(g * 16, 16)] = one
    pltpu.sync_copy(dbuf_v, deg_sh.at[pl.ds(s * DSL, DSL)])
    plsc.subcore_barrier()

    def sblk(S, _):
        sbase = s * EPT + S * SCK
        pltpu.sync_copy(edst.at[pl.ds(sbase, SCK)], edstS)
        pltpu.sync_copy(et.at[pl.ds(sbase, SCK)], etS)
        for p in range(SCP):
            for g in range(CE // 16):
                sl = pl.ds(p * CE + g * 16, 16)
                sixS2[p, pl.ds(g * 16, 16)] = etS[sl] * N + edstS[sl]
        descs = [
            pltpu.async_copy(ones_v, deg_sh.at[sixS2.at[p]], semD, add=True)
            for p in range(SCP)
        ]
        for d in descs:
            d.wait()
        return 0

    lax.fori_loop(0, NSC_D, sblk, 0)
    plsc.subcore_barrier()

    pltpu.sync_copy(deg_sh.at[pl.ds(s * DSL, DSL)], dbuf_v)

    def recip(i, _):
        sl = pl.ds(i * 16, 16)
        dbuf_v[sl] = 1.0 / jnp.maximum(dbuf_v[sl], 1.0)
        return 0

    lax.fori_loop(0, DSL // 16, recip, 0)

    @pl.when(c == 0)
    def _():
        pltpu.sync_copy(dbuf_v, stab.at[pl.ds(s * DSL, DSL)])


def _sc_degree(edst, et):
    f = pl.kernel(
        _sc_degree_body,
        out_type=jax.ShapeDtypeStruct((DEGP,), _f32),
        mesh=_mesh(),
        compiler_params=pltpu.CompilerParams(needs_layout_passes=False),
        scratch_types=[
            pltpu.VMEM_SHARED((DEGP,), _f32),
            pltpu.VMEM((DSL,), _f32),
            pltpu.VMEM((SCK,), _i32),
            pltpu.VMEM((SCK,), _i32),
            pltpu.VMEM((SCP, CE), _i32),
            pltpu.VMEM((CE,), _f32),
            pltpu.SemaphoreType.DMA,
        ],
    )
    return f(edst, et)


def _scale_rows_off(msg_v, sclS, off):
    @plsc.parallel_loop(0, CE, unroll=4)
    def srow(e):
        bc = plsc.load_gather(sclS, [jnp.broadcast_to(off + e, (16,)).astype(_i32)])
        for jj in range(8):
            msg_v[e, pl.ds(jj * 16, 16)] = msg_v[e, pl.ds(jj * 16, 16)] * bc


def _msg_superchunk(xwtab, edstS, gixS, sclS, acc_sh, dsts, msgs, semG, semW):
    ns = len(msgs)
    gds = {}
    scs = {}

    def fire(p):
        for g in range(CE // 16):
            dsts[p % ns][pl.ds(g * 16, 16)] = edstS[pl.ds(p * CE + g * 16, 16)]
        gds[p] = pltpu.async_copy(
            xwtab.at[gixS.at[pl.ds(p * CE, CE)]], msgs[p % ns], semG
        )

    for p in range(ns - 1):
        fire(p)
    for p in range(SCP):
        gds[p].wait()
        _scale_rows_off(msgs[p % ns], sclS, p * CE)
        scs[p] = pltpu.async_copy(
            msgs[p % ns], acc_sh.at[dsts[p % ns]], semW, add=True
        )
        q = p + ns - 1
        if q < SCP:
            if q - ns >= 0:
                scs[q - ns].wait()
            fire(q)
    for p in range(max(0, SCP - ns), SCP):
        scs[p].wait()


def _zero_acc(zrow_v, acc_sh, s):
    _zero_rows(zrow_v)

    def zacc(k, _):
        pltpu.sync_copy(zrow_v, acc_sh.at[pl.ds(s * NPT + k * 16, 16), :])
        return 0

    lax.fori_loop(0, NPT // 16, zacc, 0)


def _writeback(acc_sh, partial, c, s):
    pltpu.sync_copy(
        acc_sh.at[pl.ds(s * NPT, NPT), :], partial.at[c, pl.ds(s * NPT, NPT), :]
    )


def _sc_layer1_body(
    xwtab, esrc, edst, et, stab,
    partial, gidx_out, scale_out,
    acc_sh,
    zrow_v,
    srcS, edstS, etS, gixS, sixS, sclS,
    dstA, dstB, dstC, msgA, msgB, msgC, semG, semW, semS,
):
    c = lax.axis_index("c")
    s = lax.axis_index("s")
    w = c * NS + s

    _zero_acc(zrow_v, acc_sh, s)
    plsc.subcore_barrier()

    def sblk(S, _):
        sbase = w * EPW + S * SCK
        pltpu.sync_copy(esrc.at[pl.ds(sbase, SCK)], srcS)
        pltpu.sync_copy(et.at[pl.ds(sbase, SCK)], etS)
        pltpu.sync_copy(edst.at[pl.ds(sbase, SCK)], edstS)
        for i in range(SCK // 16):
            sl = pl.ds(i * 16, 16)
            t16 = etS[sl]
            gixS[sl] = (t16 + 1) * N + srcS[sl]
            sixS[sl] = t16 * N + edstS[sl]
        descs = [
            pltpu.async_copy(
                stab.at[sixS.at[pl.ds(p * CE, CE)]],
                sclS.at[pl.ds(p * CE, CE)], semS,
            )
            for p in range(SCP)
        ]
        for d in descs:
            d.wait()
        pltpu.sync_copy(gixS, gidx_out.at[pl.ds(sbase, SCK)])
        pltpu.sync_copy(sclS, scale_out.at[pl.ds(sbase, SCK)])
        _msg_superchunk(xwtab, edstS, gixS, sclS, acc_sh,
                        (dstA, dstB, dstC), (msgA, msgB, msgC), semG, semW)
        return 0

    lax.fori_loop(0, NSC, sblk, 0)
    plsc.subcore_barrier()
    _writeback(acc_sh, partial, c, s)


def _sc_layer1(xwtab_flat, esrc, edst, et, stab):
    f = pl.kernel(
        _sc_layer1_body,
        out_type=(
            jax.ShapeDtypeStruct((NC, NP, D), _f32),
            jax.ShapeDtypeStruct((E,), _i32),
            jax.ShapeDtypeStruct((E,), _f32),
        ),
        mesh=_mesh(),
        compiler_params=pltpu.CompilerParams(needs_layout_passes=False),
        scratch_types=[
            pltpu.VMEM_SHARED((NP, D), _f32),
            pltpu.VMEM((16, D), _f32),
            pltpu.VMEM((SCK,), _i32),
            pltpu.VMEM((SCK,), _i32),
            pltpu.VMEM((SCK,), _i32),
            pltpu.VMEM((SCK,), _i32),
            pltpu.VMEM((SCK,), _i32),
            pltpu.VMEM((SCK,), _f32),
            pltpu.VMEM((CE,), _i32),
            pltpu.VMEM((CE,), _i32),
            pltpu.VMEM((CE,), _i32),
            pltpu.VMEM((CE, D), _f32),
            pltpu.VMEM((CE, D), _f32),
            pltpu.VMEM((CE, D), _f32),
            pltpu.SemaphoreType.DMA,
            pltpu.SemaphoreType.DMA,
            pltpu.SemaphoreType.DMA,
        ],
    )
    return f(xwtab_flat, esrc, edst, et, stab)


def _sc_layer2_body(
    xwtab, edst, gidx, scale,
    partial,
    acc_sh, zrow_v, edstS, gixS, sclS,
    dstA, dstB, dstC, msgA, msgB, msgC, semG, semW,
):
    c = lax.axis_index("c")
    s = lax.axis_index("s")
    w = c * NS + s

    _zero_acc(zrow_v, acc_sh, s)
    plsc.subcore_barrier()

    def sblk(S, _):
        sbase = w * EPW + S * SCK
        pltpu.sync_copy(edst.at[pl.ds(sbase, SCK)], edstS)
        pltpu.sync_copy(gidx.at[pl.ds(sbase, SCK)], gixS)
        pltpu.sync_copy(scale.at[pl.ds(sbase, SCK)], sclS)
        _msg_superchunk(xwtab, edstS, gixS, sclS, acc_sh,
                        (dstA, dstB, dstC), (msgA, msgB, msgC), semG, semW)
        return 0

    lax.fori_loop(0, NSC, sblk, 0)
    plsc.subcore_barrier()
    _writeback(acc_sh, partial, c, s)


def _sc_layer2(xwtab_flat, edst, gidx, scale):
    f = pl.kernel(
        _sc_layer2_body,
        out_type=jax.ShapeDtypeStruct((NC, NP, D), _f32),
        mesh=_mesh(),
        compiler_params=pltpu.CompilerParams(needs_layout_passes=False),
        scratch_types=[
            pltpu.VMEM_SHARED((NP, D), _f32),
            pltpu.VMEM((16, D), _f32),
            pltpu.VMEM((SCK,), _i32),
            pltpu.VMEM((SCK,), _i32),
            pltpu.VMEM((SCK,), _f32),
            pltpu.VMEM((CE,), _i32),
            pltpu.VMEM((CE,), _i32),
            pltpu.VMEM((CE,), _i32),
            pltpu.VMEM((CE, D), _f32),
            pltpu.VMEM((CE, D), _f32),
            pltpu.VMEM((CE, D), _f32),
            pltpu.SemaphoreType.DMA,
            pltpu.SemaphoreType.DMA,
        ],
    )
    return f(xwtab_flat, edst, gidx, scale)


def _sc_pair_gather_body(p0, p1, s1, b1, dridx, disidx, dr_out, di_out,
                         idx_v, a_v, b_v, bbuf, sem):
    c = lax.axis_index("c")
    s = lax.axis_index("s")
    w = c * NS + s
    base = w * BPW
    pltpu.sync_copy(b1, bbuf)

    for idxsrc, out in ((dridx, dr_out), (disidx, di_out)):
        pltpu.sync_copy(idxsrc.at[pl.ds(base, BPW)], idx_v)
        da = pltpu.async_copy(p0.at[idx_v], a_v, sem)
        db = pltpu.async_copy(p1.at[idx_v], b_v, sem)
        da.wait()
        db.wait()

        @plsc.parallel_loop(0, BPW, unroll=4)
        def _(e):
            for j in range(8):
                sl = pl.ds(j * 16, 16)
                a_v[e, sl] = a_v[e, sl] + b_v[e, sl]

        pltpu.async_copy(s1.at[idx_v], b_v, sem).wait()

        @plsc.parallel_loop(0, BPW, unroll=4)
        def _(e):
            for j in range(8):
                sl = pl.ds(j * 16, 16)
                a_v[e, sl] = a_v[e, sl] + b_v[e, sl] + bbuf[sl]

        pltpu.sync_copy(a_v, out.at[pl.ds(base, BPW), :])


def _sc_pair_gather(p0, p1, s1, b1, dridx, disidx):
    f = pl.kernel(
        _sc_pair_gather_body,
        out_type=(
            jax.ShapeDtypeStruct((B, D), _f32),
            jax.ShapeDtypeStruct((B, D), _f32),
        ),
        mesh=_mesh(),
        compiler_params=pltpu.CompilerParams(needs_layout_passes=False),
        scratch_types=[
            pltpu.VMEM((BPW,), _i32),
            pltpu.VMEM((BPW, D), _f32),
            pltpu.VMEM((BPW, D), _f32),
            pltpu.VMEM((D,), _f32),
            pltpu.SemaphoreType.DMA,
        ],
    )
    return f(p0, p1, s1, b1, dridx, disidx)


def kernel(emb, rel_w0, self_w0, b0, rel_w1, self_w1, b1, pW1, pb1, pW2, pb2,
           edge_index, edge_type, drug_index, disease_index):
    w0 = jnp.concatenate([self_w0[None], rel_w0], axis=0)
    w0 = jnp.transpose(w0, (1, 0, 2)).reshape(D, R1 * D)
    w1 = jnp.concatenate([self_w1[None], rel_w1], axis=0)
    w1 = jnp.transpose(w1, (1, 0, 2)).reshape(D, R1 * D)
    esrc = edge_index[0].astype(_i32)
    edst = edge_index[1].astype(_i32)
    et = edge_type.astype(_i32)

    stab = _sc_degree(edst, et)
    xw0 = _tc_tables(emb, w0)
    part0, gidx, scale = _sc_layer1(xw0.reshape(R1 * N, D), esrc, edst, et,
                                    stab)
    xw1 = _tc_tables_fused(part0[0], part0[1], xw0[0],
                           b0.reshape(1, D), w1)
    part1 = _sc_layer2(xw1.reshape(R1 * N, D), edst, gidx, scale)
    dr, di = _sc_pair_gather(part1[0], part1[1], xw1[0], b1,
                             drug_index.astype(_i32), disease_index.astype(_i32))
    logits = _tc_pair(dr, di, pW1, pb1.reshape(1, PH), pW2, pb2.reshape(1, 1))
    return logits.reshape(B)


# speedup vs baseline: 2.1945x; 1.0364x over previous
import functools

import jax
import jax.numpy as jnp
from jax import lax
from jax.experimental import pallas as pl
from jax.experimental.pallas import tpu as pltpu
from jax.experimental.pallas import tpu_sc as plsc

N = 10000
E = 320000
D = 128
R = 4
B = 4096
PH = 128

R1 = R + 1
NC = 2
NS = 16
NW = NC * NS
EPW = E // NW
EPT = E // NS
CE = 80
NCH = EPW // CE
DCH = EPT // CE
SCK = 2000
SCP = SCK // CE
NSC = EPW // SCK
DEGP = 40960
DSL = DEGP // NS
NP = 10240
NPT = NP // NS
BN = 1000
NB = N // BN
BPW = B // NW

_f32 = jnp.float32
_i32 = jnp.int32


def _mesh():
    return plsc.VectorSubcoreMesh(
        core_axis_name="c", subcore_axis_name="s", num_cores=NC, num_subcores=NS
    )


def _mm_body(x_ref, w_ref, o_ref):
    res = jnp.dot(x_ref[...], w_ref[...], preferred_element_type=_f32)
    for r in range(R1):
        o_ref[r] = res[:, r * D:(r + 1) * D]


def _tc_tables(x, wcat):
    return pl.pallas_call(
        _mm_body,
        grid=(NB,),
        in_specs=[
            pl.BlockSpec((BN, D), lambda i: (i, 0)),
            pl.BlockSpec((D, R1 * D), lambda i: (0, 0)),
        ],
        out_specs=pl.BlockSpec((R1, BN, D), lambda i: (0, i, 0)),
        out_shape=jax.ShapeDtypeStruct((R1, N, D), _f32),
    )(x, wcat)


def _mm_fused_body(p0_ref, p1_ref, s0_ref, b_ref, w_ref, o_ref):
    x = jnp.maximum(p0_ref[0] + p1_ref[0] + s0_ref[0] + b_ref[...], 0.0)
    res = jnp.dot(x, w_ref[...], preferred_element_type=_f32)
    for r in range(R1):
        o_ref[r] = res[:, r * D:(r + 1) * D]


def _tc_tables_fused(part, xw0, b, wcat):
    return pl.pallas_call(
        _mm_fused_body,
        grid=(NB,),
        in_specs=[
            pl.BlockSpec((1, BN, D), lambda i: (0, i, 0)),
            pl.BlockSpec((1, BN, D), lambda i: (1, i, 0)),
            pl.BlockSpec((1, BN, D), lambda i: (0, i, 0)),
            pl.BlockSpec((1, D), lambda i: (0, 0)),
            pl.BlockSpec((D, R1 * D), lambda i: (0, 0)),
        ],
        out_specs=pl.BlockSpec((R1, BN, D), lambda i: (0, i, 0)),
        out_shape=jax.ShapeDtypeStruct((R1, N, D), _f32),
    )(part, part, xw0, b, wcat)


def _sum_body(p0_ref, p1_ref, s0_ref, b_ref, o_ref):
    o_ref[...] = p0_ref[...] + p1_ref[...] + s0_ref[...] + b_ref[...]


def _tc_sum(p0, p1, s0, b):
    return pl.pallas_call(
        _sum_body,
        grid=(NB,),
        in_specs=[
            pl.BlockSpec((BN, D), lambda i: (i, 0)),
            pl.BlockSpec((BN, D), lambda i: (i, 0)),
            pl.BlockSpec((BN, D), lambda i: (i, 0)),
            pl.BlockSpec((1, D), lambda i: (0, 0)),
        ],
        out_specs=pl.BlockSpec((BN, D), lambda i: (i, 0)),
        out_shape=jax.ShapeDtypeStruct((N, D), _f32),
    )(p0, p1, s0, b)


def _pair_body(dr_ref, di_ref, w1_ref, b1_ref, w2_ref, b2_ref, o_ref):
    dr = dr_ref[...]
    di = di_ref[...]
    h = (
        jnp.dot(dr, w1_ref[0:D, :], preferred_element_type=_f32)
        + jnp.dot(di, w1_ref[D : 2 * D, :], preferred_element_type=_f32)
        + jnp.dot(dr * di, w1_ref[2 * D : 3 * D, :], preferred_element_type=_f32)
        + b1_ref[...]
    )
    h = jnp.maximum(h, 0.0)
    o_ref[...] = (jnp.dot(h, w2_ref[...], preferred_element_type=_f32)
                  + b2_ref[...])[:, 0]


def _tc_pair(dr, di, w1, b1, w2, b2):
    BB = 512
    return pl.pallas_call(
        _pair_body,
        grid=(B // BB,),
        in_specs=[
            pl.BlockSpec((BB, D), lambda i: (i, 0)),
            pl.BlockSpec((BB, D), lambda i: (i, 0)),
            pl.BlockSpec((3 * D, PH), lambda i: (0, 0)),
            pl.BlockSpec((1, PH), lambda i: (0, 0)),
            pl.BlockSpec((PH, 1), lambda i: (0, 0)),
            pl.BlockSpec((1, 1), lambda i: (0, 0)),
        ],
        out_specs=pl.BlockSpec((BB,), lambda i: (i,)),
        out_shape=jax.ShapeDtypeStruct((B,), _f32),
    )(dr, di, w1, b1, w2, b2)


_Z16 = lambda: jnp.zeros((16,), _f32)


def _zero_rows(zrow_v):
    z = _Z16()

    def row(i, _):
        for jj in range(8):
            zrow_v[i, pl.ds(jj * 16, 16)] = z
        return 0

    lax.fori_loop(0, 16, row, 0)


def _scale_rows(msg_v, scl_v, count):
    def srow(e, _):
        bc = plsc.load_gather(scl_v, [jnp.broadcast_to(e, (16,)).astype(_i32)])
        for jj in range(8):
            msg_v[e, pl.ds(jj * 16, 16)] = msg_v[e, pl.ds(jj * 16, 16)] * bc
        return 0

    lax.fori_loop(0, count, srow, 0)


NSC_D = EPT // SCK


def _sc_degree_body(edst, et, stab, deg_sh, dbuf_v, edstS, etS, sixS2,
                    ones_v, semD):
    c = lax.axis_index("c")
    s = lax.axis_index("s")
    z = _Z16()
    one = jnp.ones((16,), _f32)

    def zdeg(i, _):
        dbuf_v[pl.ds(i * 16, 16)] = z
        return 0

    lax.fori_loop(0, DSL // 16, zdeg, 0)
    for g in range(CE // 16):
        ones_v[pl.ds(g * 16, 16)] = one
    pltpu.sync_copy(dbuf_v, deg_sh.at[pl.ds(s * DSL, DSL)])
    plsc.subcore_barrier()

    def sblk(S, _):
        sbase = s * EPT + S * SCK
        pltpu.sync_copy(edst.at[pl.ds(sbase, SCK)], edstS)
        pltpu.sync_copy(et.at[pl.ds(sbase, SCK)], etS)
        for p in range(SCP):
            for g in range(CE // 16):
                sl = pl.ds(p * CE + g * 16, 16)
                sixS2[p, pl.ds(g * 16, 16)] = etS[sl] * N + edstS[sl]
        descs = [
            pltpu.async_copy(ones_v, deg_sh.at[sixS2.at[p]], semD, add=True)
            for p in range(SCP)
        ]
        for d in descs:
            d.wait()
        return 0

    lax.fori_loop(0, NSC_D, sblk, 0)
    plsc.subcore_barrier()

    pltpu.sync_copy(deg_sh.at[pl.ds(s * DSL, DSL)], dbuf_v)

    def recip(i, _):
        sl = pl.ds(i * 16, 16)
        dbuf_v[sl] = 1.0 / jnp.maximum(dbuf_v[sl], 1.0)
        return 0

    lax.fori_loop(0, DSL // 16, recip, 0)

    @pl.when(c == 0)
    def _():
        pltpu.sync_copy(dbuf_v, stab.at[pl.ds(s * DSL, DSL)])


def _sc_degree(edst, et):
    f = pl.kernel(
        _sc_degree_body,
        out_type=jax.ShapeDtypeStruct((DEGP,), _f32),
        mesh=_mesh(),
        compiler_params=pltpu.CompilerParams(needs_layout_passes=False),
        scratch_types=[
            pltpu.VMEM_SHARED((DEGP,), _f32),
            pltpu.VMEM((DSL,), _f32),
            pltpu.VMEM((SCK,), _i32),
            pltpu.VMEM((SCK,), _i32),
            pltpu.VMEM((SCP, CE), _i32),
            pltpu.VMEM((CE,), _f32),
            pltpu.SemaphoreType.DMA,
        ],
    )
    return f(edst, et)


def _scale_rows_off(msg_v, sclS, off):
    @plsc.parallel_loop(0, CE, unroll=4)
    def srow(e):
        bc = plsc.load_gather(sclS, [jnp.broadcast_to(off + e, (16,)).astype(_i32)])
        for jj in range(8):
            msg_v[e, pl.ds(jj * 16, 16)] = msg_v[e, pl.ds(jj * 16, 16)] * bc


def _msg_superchunk(xwtab, edstS, gixS, sclS, acc_sh, dsts, msgs, semG, semW):
    ns = len(msgs)
    gds = {}
    scs = {}

    def fire(p):
        for g in range(CE // 16):
            dsts[p % ns][pl.ds(g * 16, 16)] = edstS[pl.ds(p * CE + g * 16, 16)]
        gds[p] = pltpu.async_copy(
            xwtab.at[gixS.at[pl.ds(p * CE, CE)]], msgs[p % ns], semG
        )

    for p in range(ns - 1):
        fire(p)
    for p in range(SCP):
        gds[p].wait()
        _scale_rows_off(msgs[p % ns], sclS, p * CE)
        scs[p] = pltpu.async_copy(
            msgs[p % ns], acc_sh.at[dsts[p % ns]], semW, add=True
        )
        q = p + ns - 1
        if q < SCP:
            if q - ns >= 0:
                scs[q - ns].wait()
            fire(q)
    for p in range(max(0, SCP - ns), SCP):
        scs[p].wait()


def _zero_acc(zrow_v, acc_sh, s):
    _zero_rows(zrow_v)

    def zacc(k, _):
        pltpu.sync_copy(zrow_v, acc_sh.at[pl.ds(s * NPT + k * 16, 16), :])
        return 0

    lax.fori_loop(0, NPT // 16, zacc, 0)


def _writeback(acc_sh, partial, c, s):
    pltpu.sync_copy(
        acc_sh.at[pl.ds(s * NPT, NPT), :], partial.at[c, pl.ds(s * NPT, NPT), :]
    )


def _sc_layer1_body(
    xwtab, esrc, edst, et, stab,
    partial, gidx_out, scale_out,
    acc_sh,
    zrow_v,
    srcS, edstS, etS, gixS, sixS, sclS,
    dstA, dstB, dstC, msgA, msgB, msgC, semG, semW, semS,
):
    c = lax.axis_index("c")
    s = lax.axis_index("s")
    w = c * NS + s

    _zero_acc(zrow_v, acc_sh, s)
    plsc.subcore_barrier()

    def sblk(S, _):
        sbase = w * EPW + S * SCK
        pltpu.sync_copy(esrc.at[pl.ds(sbase, SCK)], srcS)
        pltpu.sync_copy(et.at[pl.ds(sbase, SCK)], etS)
        pltpu.sync_copy(edst.at[pl.ds(sbase, SCK)], edstS)
        for i in range(SCK // 16):
            sl = pl.ds(i * 16, 16)
            t16 = etS[sl]
            gixS[sl] = (t16 + 1) * N + srcS[sl]
            sixS[sl] = t16 * N + edstS[sl]
        descs = [
            pltpu.async_copy(
                stab.at[sixS.at[pl.ds(p * CE, CE)]],
                sclS.at[pl.ds(p * CE, CE)], semS,
            )
            for p in range(SCP)
        ]
        for d in descs:
            d.wait()
        pltpu.sync_copy(gixS, gidx_out.at[pl.ds(sbase, SCK)])
        pltpu.sync_copy(sclS, scale_out.at[pl.ds(sbase, SCK)])
        _msg_superchunk(xwtab, edstS, gixS, sclS, acc_sh,
                        (dstA, dstB, dstC), (msgA, msgB, msgC), semG, semW)
        return 0

    lax.fori_loop(0, NSC, sblk, 0)
    plsc.subcore_barrier()
    _writeback(acc_sh, partial, c, s)


def _sc_layer1(xwtab_flat, esrc, edst, et, stab):
    f = pl.kernel(
        _sc_layer1_body,
        out_type=(
            jax.ShapeDtypeStruct((NC, NP, D), _f32),
            jax.ShapeDtypeStruct((E,), _i32),
            jax.ShapeDtypeStruct((E,), _f32),
        ),
        mesh=_mesh(),
        compiler_params=pltpu.CompilerParams(needs_layout_passes=False),
        scratch_types=[
            pltpu.VMEM_SHARED((NP, D), _f32),
            pltpu.VMEM((16, D), _f32),
            pltpu.VMEM((SCK,), _i32),
            pltpu.VMEM((SCK,), _i32),
            pltpu.VMEM((SCK,), _i32),
            pltpu.VMEM((SCK,), _i32),
            pltpu.VMEM((SCK,), _i32),
            pltpu.VMEM((SCK,), _f32),
            pltpu.VMEM((CE,), _i32),
            pltpu.VMEM((CE,), _i32),
            pltpu.VMEM((CE,), _i32),
            pltpu.VMEM((CE, D), _f32),
            pltpu.VMEM((CE, D), _f32),
            pltpu.VMEM((CE, D), _f32),
            pltpu.SemaphoreType.DMA,
            pltpu.SemaphoreType.DMA,
            pltpu.SemaphoreType.DMA,
        ],
    )
    return f(xwtab_flat, esrc, edst, et, stab)


def _sc_layer2_body(
    xwtab, edst, gidx, scale,
    partial,
    acc_sh, zrow_v, edstS, gixS, sclS,
    dstA, dstB, dstC, msgA, msgB, msgC, semG, semW,
):
    c = lax.axis_index("c")
    s = lax.axis_index("s")
    w = c * NS + s

    _zero_acc(zrow_v, acc_sh, s)
    plsc.subcore_barrier()

    def sblk(S, _):
        sbase = w * EPW + S * SCK
        pltpu.sync_copy(edst.at[pl.ds(sbase, SCK)], edstS)
        pltpu.sync_copy(gidx.at[pl.ds(sbase, SCK)], gixS)
        pltpu.sync_copy(scale.at[pl.ds(sbase, SCK)], sclS)
        _msg_superchunk(xwtab, edstS, gixS, sclS, acc_sh,
                        (dstA, dstB, dstC), (msgA, msgB, msgC), semG, semW)
        return 0

    lax.fori_loop(0, NSC, sblk, 0)
    plsc.subcore_barrier()
    _writeback(acc_sh, partial, c, s)


def _sc_layer2(xwtab_flat, edst, gidx, scale):
    f = pl.kernel(
        _sc_layer2_body,
        out_type=jax.ShapeDtypeStruct((NC, NP, D), _f32),
        mesh=_mesh(),
        compiler_params=pltpu.CompilerParams(needs_layout_passes=False),
        scratch_types=[
            pltpu.VMEM_SHARED((NP, D), _f32),
            pltpu.VMEM((16, D), _f32),
            pltpu.VMEM((SCK,), _i32),
            pltpu.VMEM((SCK,), _i32),
            pltpu.VMEM((SCK,), _f32),
            pltpu.VMEM((CE,), _i32),
            pltpu.VMEM((CE,), _i32),
            pltpu.VMEM((CE,), _i32),
            pltpu.VMEM((CE, D), _f32),
            pltpu.VMEM((CE, D), _f32),
            pltpu.VMEM((CE, D), _f32),
            pltpu.SemaphoreType.DMA,
            pltpu.SemaphoreType.DMA,
        ],
    )
    return f(xwtab_flat, edst, gidx, scale)


def _sc_pair_gather_body(pflat, s1, b1, dridx, disidx, dr_out, di_out,
                         idx_v, idx2_v, a_v, b_v, bbuf, sem):
    c = lax.axis_index("c")
    s = lax.axis_index("s")
    w = c * NS + s
    base = w * BPW
    pltpu.sync_copy(b1, bbuf)

    for idxsrc, out in ((dridx, dr_out), (disidx, di_out)):
        pltpu.sync_copy(idxsrc.at[pl.ds(base, BPW)], idx_v)
        for g in range(BPW // 16):
            sl = pl.ds(g * 16, 16)
            idx2_v[sl] = idx_v[sl] + NP
        da = pltpu.async_copy(pflat.at[idx_v], a_v, sem)
        db = pltpu.async_copy(pflat.at[idx2_v], b_v, sem)
        da.wait()
        db.wait()

        @plsc.parallel_loop(0, BPW, unroll=4)
        def _(e):
            for j in range(8):
                sl = pl.ds(j * 16, 16)
                a_v[e, sl] = a_v[e, sl] + b_v[e, sl]

        pltpu.async_copy(s1.at[idx_v], b_v, sem).wait()

        @plsc.parallel_loop(0, BPW, unroll=4)
        def _(e):
            for j in range(8):
                sl = pl.ds(j * 16, 16)
                a_v[e, sl] = a_v[e, sl] + b_v[e, sl] + bbuf[sl]

        pltpu.sync_copy(a_v, out.at[pl.ds(base, BPW), :])


def _sc_pair_gather(pflat, s1, b1, dridx, disidx):
    f = pl.kernel(
        _sc_pair_gather_body,
        out_type=(
            jax.ShapeDtypeStruct((B, D), _f32),
            jax.ShapeDtypeStruct((B, D), _f32),
        ),
        mesh=_mesh(),
        compiler_params=pltpu.CompilerParams(needs_layout_passes=False),
        scratch_types=[
            pltpu.VMEM((BPW,), _i32),
            pltpu.VMEM((BPW,), _i32),
            pltpu.VMEM((BPW, D), _f32),
            pltpu.VMEM((BPW, D), _f32),
            pltpu.VMEM((D,), _f32),
            pltpu.SemaphoreType.DMA,
        ],
    )
    return f(pflat, s1, b1, dridx, disidx)


def kernel(emb, rel_w0, self_w0, b0, rel_w1, self_w1, b1, pW1, pb1, pW2, pb2,
           edge_index, edge_type, drug_index, disease_index):
    w0 = jnp.concatenate([self_w0[None], rel_w0], axis=0)
    w0 = jnp.transpose(w0, (1, 0, 2)).reshape(D, R1 * D)
    w1 = jnp.concatenate([self_w1[None], rel_w1], axis=0)
    w1 = jnp.transpose(w1, (1, 0, 2)).reshape(D, R1 * D)
    esrc = edge_index[0].astype(_i32)
    edst = edge_index[1].astype(_i32)
    et = edge_type.astype(_i32)

    stab = _sc_degree(edst, et)
    xw0 = _tc_tables(emb, w0)
    part0, gidx, scale = _sc_layer1(xw0.reshape(R1 * N, D), esrc, edst, et,
                                    stab)
    xw1 = _tc_tables_fused(part0, xw0, b0.reshape(1, D), w1)
    part1 = _sc_layer2(xw1.reshape(R1 * N, D), edst, gidx, scale)
    dr, di = _sc_pair_gather(part1.reshape(NC * NP, D), xw1.reshape(R1 * N, D), b1,
                             drug_index.astype(_i32), disease_index.astype(_i32))
    logits = _tc_pair(dr, di, pW1, pb1.reshape(1, PH), pW2, pb2.reshape(1, 1))
    return logits
